# Initial kernel scaffold; baseline (speedup 1.0000x reference)
import jax
import jax.numpy as jnp
from jax.experimental import pallas as pl


def kernel(x, edge_index, W1, as1, ad1, b1, W2, as2, ad2, b2, W3, as3, ad3, b3):
    raise NotImplementedError("write your pallas kernel here")


# speedup vs baseline: 14.1240x; 14.1240x over previous
import functools

import jax
import jax.numpy as jnp
from jax import lax
from jax.experimental import pallas as pl
from jax.experimental.pallas import tpu as pltpu
from jax.experimental.pallas import tpu_sc as plsc

N = 10000
E = 320000
D = 128

NC = 2
NS = 16
NW = NC * NS
CH = 80
B = 128
SCK = 8
NSUP = CH // SCK
EPW = CH * B
E_PAD = NW * EPW
EVALID = E // NW
NPAD = 10240


def _splat_i32(v):
    return lax.full((16,), v, jnp.int32)


def _sc_body(h_hbm, es_hbm, ed_hbm, src_hbm, dst_hbm, num_out, den_out,
             sidx_v, didx_v, ex_c, es_v, ed_v, rows_v, zden_v,
             num_acc, den_acc, sem):
    c = lax.axis_index("c")
    s = lax.axis_index("s")
    wid = s * NC + c

    pltpu.sync_copy(es_hbm, es_v)
    pltpu.sync_copy(ed_hbm, ed_v)

    zero16 = jnp.zeros((16,), jnp.float32)

    def _zrow(j, _):
        for k in range(8):
            rows_v[j, pl.ds(k * 16, 16)] = zero16
        return 0

    lax.fori_loop(0, B, _zrow, 0)
    for g in range(64):
        zden_v[pl.ds(g * 16, 16)] = zero16

    for i in range(4):
        pltpu.sync_copy(rows_v.at[pl.ds(0, 128)],
                        num_acc.at[pl.ds(s * 624 + i * 128, 128)])
    pltpu.sync_copy(rows_v.at[pl.ds(0, 112)],
                    num_acc.at[pl.ds(s * 624 + 512, 112)])

    @pl.when(s == 0)
    def _():
        pltpu.sync_copy(rows_v.at[pl.ds(0, 16)],
                        num_acc.at[pl.ds(9984, 16)])

    @pl.when(s < 10)
    def _():
        pltpu.sync_copy(zden_v, den_acc.at[pl.ds(s * 1024, 1024)])

    plsc.subcore_barrier()

    def _super(g, _):
        r0 = wid * CH + g * SCK
        pltpu.sync_copy(src_hbm.at[pl.ds(r0, SCK)], sidx_v)
        pltpu.sync_copy(dst_hbm.at[pl.ds(r0, SCK)], didx_v)

        def _chunk(k, _):
            base = wid * EPW + (g * SCK + k) * B
            for grp in range(8):
                sl = pl.ds(grp * 16, 16)
                si = sidx_v[k, sl]
                di = didx_v[k, sl]
                t = (plsc.load_gather(es_v, [si])
                     + plsc.load_gather(ed_v, [di]))
                e = jnp.maximum(t, t * jnp.float32(0.2))
                ex = jnp.exp(e)
                fac = lax.select(base + grp * 16 < E,
                                 jnp.float32(1), jnp.float32(0))
                ex_c[k, sl] = ex * lax.full((16,), fac, jnp.float32)
            pltpu.async_copy(h_hbm.at[sidx_v.at[k]], rows_v, sem).wait()

            def _scale_row(r, _):
                exs = plsc.load_gather(ex_c, [_splat_i32(k), _splat_i32(r)])
                for q in range(8):
                    sl = pl.ds(q * 16, 16)
                    rows_v[r, sl] = rows_v[r, sl] * exs
                return 0

            lax.fori_loop(0, B, _scale_row, 0)
            pltpu.sync_copy(rows_v, num_acc.at[didx_v.at[k]], add=True)
            pltpu.sync_copy(ex_c.at[k], den_acc.at[didx_v.at[k]], add=True)
            return 0

        lax.fori_loop(0, SCK, _chunk, 0)
        return 0

    lax.fori_loop(0, NSUP, _super, 0)

    plsc.subcore_barrier()
    for i in range(4):
        r0 = s * 624 + i * 128
        pltpu.sync_copy(num_acc.at[pl.ds(r0, 128)],
                        num_out.at[c, pl.ds(r0, 128)])
    pltpu.sync_copy(num_acc.at[pl.ds(s * 624 + 512, 112)],
                    num_out.at[c, pl.ds(s * 624 + 512, 112)])

    @pl.when(s == 0)
    def _():
        pltpu.sync_copy(num_acc.at[pl.ds(9984, 16)],
                        num_out.at[c, pl.ds(9984, 16)])

    @pl.when(s < 10)
    def _():
        pltpu.sync_copy(den_acc.at[pl.ds(s * 1024, 1024)],
                        den_out.at[pl.ds(c * NPAD + s * 1024, 1024)])


@functools.cache
def _make_sc_layer():
    return pl.kernel(
        _sc_body,
        out_type=(jax.ShapeDtypeStruct((NC, N, D), jnp.float32),
                  jax.ShapeDtypeStruct((NC * NPAD,), jnp.float32)),
        mesh=plsc.VectorSubcoreMesh(core_axis_name="c", subcore_axis_name="s",
                                    num_cores=NC, num_subcores=NS),
        scratch_types=[
            pltpu.VMEM((SCK, B), jnp.int32),
            pltpu.VMEM((SCK, B), jnp.int32),
            pltpu.VMEM((SCK, B), jnp.float32),
            pltpu.VMEM((N,), jnp.float32),
            pltpu.VMEM((N,), jnp.float32),
            pltpu.VMEM((B, D), jnp.float32),
            pltpu.VMEM((1024,), jnp.float32),
            pltpu.VMEM_SHARED((N, D), jnp.float32),
            pltpu.VMEM_SHARED((NPAD,), jnp.float32),
            pltpu.SemaphoreType.DMA,
        ],
        compiler_params=pltpu.CompilerParams(use_tc_tiling_on_sc=False,
                                             needs_layout_passes=False),
    )


def _sc_layer(h, es, ed, src_p, dst_p):
    num_p, den_flat = _make_sc_layer()(h, es, ed, src_p, dst_p)
    return num_p, den_flat.reshape(NC, NPAD, 1)


_BLK = 1000
_GRID = N // _BLK


def _tc_first_body(x_ref, w_ref, as_ref, ad_ref, h_ref, es_ref, ed_ref):
    h = jnp.dot(x_ref[...], w_ref[...], preferred_element_type=jnp.float32)
    h_ref[...] = h
    es_ref[...] = (h @ as_ref[...])[:, None]
    ed_ref[...] = (h @ ad_ref[...])[:, None]


def _tc_mid_body(np_ref, dp_ref, b_ref, w_ref, as_ref, ad_ref,
                 h_ref, es_ref, ed_ref):
    num = np_ref[0] + np_ref[1]
    den = dp_ref[0, :, 0] + dp_ref[1, :, 0]
    x = jnp.maximum(num / (den + jnp.float32(1e-16))[:, None]
                    + b_ref[...][None, :], 0.0)
    h = jnp.dot(x, w_ref[...], preferred_element_type=jnp.float32)
    h_ref[...] = h
    es_ref[...] = (h @ as_ref[...])[:, None]
    ed_ref[...] = (h @ ad_ref[...])[:, None]


def _tc_final_body(np_ref, dp_ref, b_ref, o_ref):
    num = np_ref[0] + np_ref[1]
    den = dp_ref[0, :, 0] + dp_ref[1, :, 0]
    o_ref[...] = (num / (den + jnp.float32(1e-16))[:, None]
                  + b_ref[...][None, :])


_vec_spec = pl.BlockSpec((128,), lambda i: (0,))
_w_spec = pl.BlockSpec((D, D), lambda i: (0, 0))
_den_spec = pl.BlockSpec((NC, _BLK, 1), lambda i: (0, i, 0))
_h_out = [jax.ShapeDtypeStruct((N, D), jnp.float32),
          jax.ShapeDtypeStruct((N, 1), jnp.float32),
          jax.ShapeDtypeStruct((N, 1), jnp.float32)]
_h_specs = [pl.BlockSpec((_BLK, D), lambda i: (i, 0)),
            pl.BlockSpec((_BLK, 1), lambda i: (i, 0)),
            pl.BlockSpec((_BLK, 1), lambda i: (i, 0))]


def _tc_first(x, W, a_s, a_d):
    return pl.pallas_call(
        _tc_first_body,
        grid=(_GRID,),
        in_specs=[pl.BlockSpec((_BLK, D), lambda i: (i, 0)),
                  _w_spec, _vec_spec, _vec_spec],
        out_specs=_h_specs,
        out_shape=_h_out,
    )(x, W, a_s, a_d)


def _tc_mid(num_p, den_p, b, W, a_s, a_d):
    return pl.pallas_call(
        _tc_mid_body,
        grid=(_GRID,),
        in_specs=[pl.BlockSpec((NC, _BLK, D), lambda i: (0, i, 0)),
                  _den_spec,
                  _vec_spec, _w_spec, _vec_spec, _vec_spec],
        out_specs=_h_specs,
        out_shape=_h_out,
    )(num_p, den_p, b, W, a_s, a_d)


def _tc_final(num_p, den_p, b):
    return pl.pallas_call(
        _tc_final_body,
        grid=(_GRID,),
        in_specs=[pl.BlockSpec((NC, _BLK, D), lambda i: (0, i, 0)),
                  _den_spec,
                  _vec_spec],
        out_specs=pl.BlockSpec((_BLK, D), lambda i: (i, 0)),
        out_shape=jax.ShapeDtypeStruct((N, D), jnp.float32),
    )(num_p, den_p, b)


def kernel(x, edge_index, W1, as1, ad1, b1, W2, as2, ad2, b2,
           W3, as3, ad3, b3):
    ei = edge_index.astype(jnp.int32)
    pad = E_PAD - E
    src_p = jnp.concatenate(
        [ei[0], jnp.zeros((pad,), jnp.int32)]).reshape(NW * CH, B)
    dst_p = jnp.concatenate(
        [ei[1], jnp.zeros((pad,), jnp.int32)]).reshape(NW * CH, B)

    h, es, ed = _tc_first(x, W1, as1, ad1)
    num_p, den_p = _sc_layer(h, es.reshape(N), ed.reshape(N), src_p, dst_p)
    h, es, ed = _tc_mid(num_p, den_p, b1, W2, as2, ad2)
    num_p, den_p = _sc_layer(h, es.reshape(N), ed.reshape(N), src_p, dst_p)
    h, es, ed = _tc_mid(num_p, den_p, b2, W3, as3, ad3)
    num_p, den_p = _sc_layer(h, es.reshape(N), ed.reshape(N), src_p, dst_p)
    return _tc_final(num_p, den_p, b3)


# speedup vs baseline: 17.2968x; 1.2246x over previous
import functools

import jax
import jax.numpy as jnp
from jax import lax
from jax.experimental import pallas as pl
from jax.experimental.pallas import tpu as pltpu
from jax.experimental.pallas import tpu_sc as plsc

N = 10000
E = 320000
D = 128

NC = 2
NS = 16
NW = NC * NS
CH = 80
B = 128
SCK = 8
NSUP = CH // SCK
EPW = CH * B
E_PAD = NW * EPW
EVALID = E // NW
NPAD = 10240


def _splat_i32(v):
    return lax.full((16,), v, jnp.int32)


def _sc_body(h_hbm, es_hbm, ed_hbm, src_hbm, dst_hbm, num_out, den_out,
             ex_v, zden_v, num_acc, den_acc,
             sem_g0, sem_g1, sem_s0, sem_s1):
    c = lax.axis_index("c")
    s = lax.axis_index("s")
    wid = s * NC + c
    zero16 = jnp.zeros((16,), jnp.float32)

    def _phase_a(es_v, ed_v, sidx_v, didx_v):
        pltpu.sync_copy(es_hbm, es_v)
        pltpu.sync_copy(ed_hbm, ed_v)

        def _super_a(g, _):
            r0 = wid * CH + g * SCK
            pltpu.sync_copy(src_hbm.at[pl.ds(r0, SCK)], sidx_v)
            pltpu.sync_copy(dst_hbm.at[pl.ds(r0, SCK)], didx_v)
            for k in range(SCK):
                base = (r0 + k) * B
                for grp in range(8):
                    sl = pl.ds(grp * 16, 16)
                    t = (plsc.load_gather(es_v, [sidx_v[k, sl]])
                         + plsc.load_gather(ed_v, [didx_v[k, sl]]))
                    e = jnp.maximum(t, t * jnp.float32(0.2))
                    ex = jnp.exp(e)
                    fac = lax.select(base + grp * 16 < E,
                                     jnp.float32(1), jnp.float32(0))
                    ex_v[g * SCK + k, sl] = ex * lax.full((16,), fac,
                                                          jnp.float32)
            return 0

        lax.fori_loop(0, NSUP, _super_a, 0)

    pl.run_scoped(_phase_a,
                  pltpu.VMEM((N,), jnp.float32),
                  pltpu.VMEM((N,), jnp.float32),
                  pltpu.VMEM((SCK, B), jnp.int32),
                  pltpu.VMEM((SCK, B), jnp.int32))

    def _phase_b(rows0, rows1, sidx_v, didx_v):
        def _zrow(j, _):
            for q in range(8):
                rows0[j, pl.ds(q * 16, 16)] = zero16
            return 0

        lax.fori_loop(0, B, _zrow, 0)
        for g2 in range(64):
            zden_v[pl.ds(g2 * 16, 16)] = zero16

        for i in range(4):
            pltpu.sync_copy(rows0.at[pl.ds(0, 128)],
                            num_acc.at[pl.ds(s * 624 + i * 128, 128)])
        pltpu.sync_copy(rows0.at[pl.ds(0, 112)],
                        num_acc.at[pl.ds(s * 624 + 512, 112)])

        @pl.when(s == 0)
        def _():
            pltpu.sync_copy(rows0.at[pl.ds(0, 16)],
                            num_acc.at[pl.ds(9984, 16)])

        @pl.when(s < 10)
        def _():
            pltpu.sync_copy(zden_v, den_acc.at[pl.ds(s * 1024, 1024)])

        plsc.subcore_barrier()

        rows = (rows0, rows1)
        sems_g = (sem_g0, sem_g1)
        sems_s = (sem_s0, sem_s1)

        def _scale(buf, cg):
            def _srow(r, _):
                exs = plsc.load_gather(ex_v, [_splat_i32(cg), _splat_i32(r)])
                for q in range(8):
                    sl = pl.ds(q * 16, 16)
                    buf[r, sl] = buf[r, sl] * exs
                return 0

            lax.fori_loop(0, B, _srow, 0)

        def _drain(p, k):
            pltpu.make_async_copy(rows[p], num_acc.at[didx_v.at[k]],
                                  sems_s[p]).wait()
            pltpu.make_async_copy(ex_v.at[0], den_acc.at[didx_v.at[k]],
                                  sems_s[p]).wait()

        def _process(g, k):
            p = k % 2
            cg = g * SCK + k
            pltpu.make_async_copy(h_hbm.at[sidx_v.at[k]], rows[p],
                                  sems_g[p]).wait()
            _scale(rows[p], cg)
            pltpu.async_copy(rows[p], num_acc.at[didx_v.at[k]], sems_s[p],
                             add=True)
            pltpu.async_copy(ex_v.at[cg], den_acc.at[didx_v.at[k]],
                             sems_s[p], add=True)

        def _super_b(g, _):
            @pl.when(g >= 1)
            def _():
                _drain(0, SCK - 2)
                _drain(1, SCK - 1)

            r0 = wid * CH + g * SCK
            pltpu.sync_copy(src_hbm.at[pl.ds(r0, SCK)], sidx_v)
            pltpu.sync_copy(dst_hbm.at[pl.ds(r0, SCK)], didx_v)
            for k in range(SCK):
                if k >= 2:
                    _drain(k % 2, k)
                pltpu.async_copy(h_hbm.at[sidx_v.at[k]], rows[k % 2],
                                 sems_g[k % 2])
                if k >= 1:
                    _process(g, k - 1)
            _process(g, SCK - 1)
            return 0

        lax.fori_loop(0, NSUP, _super_b, 0)

        _drain(0, SCK - 2)
        _drain(1, SCK - 1)

    pl.run_scoped(_phase_b,
                  pltpu.VMEM((B, D), jnp.float32),
                  pltpu.VMEM((B, D), jnp.float32),
                  pltpu.VMEM((SCK, B), jnp.int32),
                  pltpu.VMEM((SCK, B), jnp.int32))

    plsc.subcore_barrier()
    for i in range(4):
        r0 = s * 624 + i * 128
        pltpu.sync_copy(num_acc.at[pl.ds(r0, 128)],
                        num_out.at[c, pl.ds(r0, 128)])
    pltpu.sync_copy(num_acc.at[pl.ds(s * 624 + 512, 112)],
                    num_out.at[c, pl.ds(s * 624 + 512, 112)])

    @pl.when(s == 0)
    def _():
        pltpu.sync_copy(num_acc.at[pl.ds(9984, 16)],
                        num_out.at[c, pl.ds(9984, 16)])

    @pl.when(s < 10)
    def _():
        pltpu.sync_copy(den_acc.at[pl.ds(s * 1024, 1024)],
                        den_out.at[pl.ds(c * NPAD + s * 1024, 1024)])


@functools.cache
def _make_sc_layer():
    return pl.kernel(
        _sc_body,
        out_type=(jax.ShapeDtypeStruct((NC, N, D), jnp.float32),
                  jax.ShapeDtypeStruct((NC * NPAD,), jnp.float32)),
        mesh=plsc.VectorSubcoreMesh(core_axis_name="c", subcore_axis_name="s",
                                    num_cores=NC, num_subcores=NS),
        scratch_types=[
            pltpu.VMEM((CH, B), jnp.float32),
            pltpu.VMEM((1024,), jnp.float32),
            pltpu.VMEM_SHARED((N, D), jnp.float32),
            pltpu.VMEM_SHARED((NPAD,), jnp.float32),
            pltpu.SemaphoreType.DMA,
            pltpu.SemaphoreType.DMA,
            pltpu.SemaphoreType.DMA,
            pltpu.SemaphoreType.DMA,
        ],
        compiler_params=pltpu.CompilerParams(use_tc_tiling_on_sc=False,
                                             needs_layout_passes=False),
    )


def _sc_layer(h, es, ed, src_p, dst_p):
    num_p, den_flat = _make_sc_layer()(h, es, ed, src_p, dst_p)
    return num_p, den_flat.reshape(NC, NPAD, 1)


_BLK = 1000
_GRID = N // _BLK


def _tc_first_body(x_ref, w_ref, as_ref, ad_ref, h_ref, es_ref, ed_ref):
    h = jnp.dot(x_ref[...], w_ref[...], preferred_element_type=jnp.float32)
    h_ref[...] = h
    es_ref[...] = (h @ as_ref[...])[:, None]
    ed_ref[...] = (h @ ad_ref[...])[:, None]


def _tc_mid_body(np_ref, dp_ref, b_ref, w_ref, as_ref, ad_ref,
                 h_ref, es_ref, ed_ref):
    num = np_ref[0] + np_ref[1]
    den = dp_ref[0, :, 0] + dp_ref[1, :, 0]
    x = jnp.maximum(num / (den + jnp.float32(1e-16))[:, None]
                    + b_ref[...][None, :], 0.0)
    h = jnp.dot(x, w_ref[...], preferred_element_type=jnp.float32)
    h_ref[...] = h
    es_ref[...] = (h @ as_ref[...])[:, None]
    ed_ref[...] = (h @ ad_ref[...])[:, None]


def _tc_final_body(np_ref, dp_ref, b_ref, o_ref):
    num = np_ref[0] + np_ref[1]
    den = dp_ref[0, :, 0] + dp_ref[1, :, 0]
    o_ref[...] = (num / (den + jnp.float32(1e-16))[:, None]
                  + b_ref[...][None, :])


_vec_spec = pl.BlockSpec((128,), lambda i: (0,))
_w_spec = pl.BlockSpec((D, D), lambda i: (0, 0))
_den_spec = pl.BlockSpec((NC, _BLK, 1), lambda i: (0, i, 0))
_h_out = [jax.ShapeDtypeStruct((N, D), jnp.float32),
          jax.ShapeDtypeStruct((N, 1), jnp.float32),
          jax.ShapeDtypeStruct((N, 1), jnp.float32)]
_h_specs = [pl.BlockSpec((_BLK, D), lambda i: (i, 0)),
            pl.BlockSpec((_BLK, 1), lambda i: (i, 0)),
            pl.BlockSpec((_BLK, 1), lambda i: (i, 0))]


def _tc_first(x, W, a_s, a_d):
    return pl.pallas_call(
        _tc_first_body,
        grid=(_GRID,),
        in_specs=[pl.BlockSpec((_BLK, D), lambda i: (i, 0)),
                  _w_spec, _vec_spec, _vec_spec],
        out_specs=_h_specs,
        out_shape=_h_out,
    )(x, W, a_s, a_d)


def _tc_mid(num_p, den_p, b, W, a_s, a_d):
    return pl.pallas_call(
        _tc_mid_body,
        grid=(_GRID,),
        in_specs=[pl.BlockSpec((NC, _BLK, D), lambda i: (0, i, 0)),
                  _den_spec,
                  _vec_spec, _w_spec, _vec_spec, _vec_spec],
        out_specs=_h_specs,
        out_shape=_h_out,
    )(num_p, den_p, b, W, a_s, a_d)


def _tc_final(num_p, den_p, b):
    return pl.pallas_call(
        _tc_final_body,
        grid=(_GRID,),
        in_specs=[pl.BlockSpec((NC, _BLK, D), lambda i: (0, i, 0)),
                  _den_spec,
                  _vec_spec],
        out_specs=pl.BlockSpec((_BLK, D), lambda i: (i, 0)),
        out_shape=jax.ShapeDtypeStruct((N, D), jnp.float32),
    )(num_p, den_p, b)


def kernel(x, edge_index, W1, as1, ad1, b1, W2, as2, ad2, b2,
           W3, as3, ad3, b3):
    ei = edge_index.astype(jnp.int32)
    pad = E_PAD - E
    src_p = jnp.concatenate(
        [ei[0], jnp.zeros((pad,), jnp.int32)]).reshape(NW * CH, B)
    dst_p = jnp.concatenate(
        [ei[1], jnp.zeros((pad,), jnp.int32)]).reshape(NW * CH, B)

    h, es, ed = _tc_first(x, W1, as1, ad1)
    num_p, den_p = _sc_layer(h, es.reshape(N), ed.reshape(N), src_p, dst_p)
    h, es, ed = _tc_mid(num_p, den_p, b1, W2, as2, ad2)
    num_p, den_p = _sc_layer(h, es.reshape(N), ed.reshape(N), src_p, dst_p)
    h, es, ed = _tc_mid(num_p, den_p, b2, W3, as3, ad3)
    num_p, den_p = _sc_layer(h, es.reshape(N), ed.reshape(N), src_p, dst_p)
    return _tc_final(num_p, den_p, b3)


# speedup vs baseline: 26.7624x; 1.5472x over previous
import functools

import jax
import jax.numpy as jnp
from jax import lax
from jax.experimental import pallas as pl
from jax.experimental.pallas import tpu as pltpu
from jax.experimental.pallas import tpu_sc as plsc

N = 10000
E = 320000
D = 128

NC = 2
NS = 16
NW = NC * NS
CH = 80
B = 128
SCK = 8
NSUP = CH // SCK
EPW = CH * B
E_PAD = NW * EPW
EVALID = E // NW
NPAD = 10240


def _splat_i32(v):
    return lax.full((16,), v, jnp.int32)


def _sc_body(h_hbm, es_hbm, ed_hbm, src_hbm, dst_hbm, num_out, den_out,
             ex_v, zden_v, num_acc, den_acc,
             sem_g0, sem_g1, sem_s0):
    c = lax.axis_index("c")
    s = lax.axis_index("s")
    wid = s * NC + c
    zero16 = jnp.zeros((16,), jnp.float32)

    def _phase_a(es_v, ed_v, sidx_v, didx_v):
        pltpu.sync_copy(es_hbm, es_v)
        pltpu.sync_copy(ed_hbm, ed_v)

        def _super_a(g, _):
            r0 = wid * CH + g * SCK
            pltpu.sync_copy(src_hbm.at[pl.ds(r0, SCK)], sidx_v)
            pltpu.sync_copy(dst_hbm.at[pl.ds(r0, SCK)], didx_v)
            for k in range(SCK):
                base = (r0 + k) * B
                for grp in range(8):
                    sl = pl.ds(grp * 16, 16)
                    t = (plsc.load_gather(es_v, [sidx_v[k, sl]])
                         + plsc.load_gather(ed_v, [didx_v[k, sl]]))
                    e = jnp.maximum(t, t * jnp.float32(0.2))
                    ex = jnp.exp(e)
                    fac = lax.select(base + grp * 16 < E,
                                     jnp.float32(1), jnp.float32(0))
                    ex_v[g * SCK + k, sl] = ex * lax.full((16,), fac,
                                                          jnp.float32)
            return 0

        lax.fori_loop(0, NSUP, _super_a, 0)

    pl.run_scoped(_phase_a,
                  pltpu.VMEM((N,), jnp.float32),
                  pltpu.VMEM((N,), jnp.float32),
                  pltpu.VMEM((SCK, B), jnp.int32),
                  pltpu.VMEM((SCK, B), jnp.int32))

    def _phase_b(rows_bf0, rows_bf1, rows_f, sidx_v, didx_v):
        iota16 = lax.iota(jnp.int32, 16)
        ev_idx = [q * 32 + 2 * iota16 for q in range(4)]
        od_idx = [q * 32 + 1 + 2 * iota16 for q in range(4)]
        himask = jnp.full((16,), -65536, jnp.int32)

        def _zrow(j, _):
            for q in range(8):
                rows_f[j, pl.ds(q * 16, 16)] = zero16
            return 0

        lax.fori_loop(0, B, _zrow, 0)
        for g2 in range(64):
            zden_v[pl.ds(g2 * 16, 16)] = zero16

        for i in range(4):
            pltpu.sync_copy(rows_f.at[pl.ds(0, 128)],
                            num_acc.at[pl.ds(s * 624 + i * 128, 128)])
        pltpu.sync_copy(rows_f.at[pl.ds(0, 112)],
                        num_acc.at[pl.ds(s * 624 + 512, 112)])

        @pl.when(s == 0)
        def _():
            pltpu.sync_copy(rows_f.at[pl.ds(0, 16)],
                            num_acc.at[pl.ds(9984, 16)])

        @pl.when(s < 10)
        def _():
            pltpu.sync_copy(zden_v, den_acc.at[pl.ds(s * 1024, 1024)])

        plsc.subcore_barrier()

        rows_bf = (rows_bf0, rows_bf1)
        sems_g = (sem_g0, sem_g1)

        def _scale(p, cg):
            def _srow(r, _):
                exs = plsc.load_gather(ex_v, [_splat_i32(cg), _splat_i32(r)])
                rsp = _splat_i32(r)
                for q in range(4):
                    w = rows_bf[p][r, pl.ds(q * 32, 32)]
                    wi = plsc.bitcast(w, jnp.int32)
                    fe = plsc.bitcast(lax.shift_left(wi, 16), jnp.float32)
                    fo = plsc.bitcast(lax.bitwise_and(wi, himask),
                                      jnp.float32)
                    plsc.store_scatter(rows_f, [rsp, ev_idx[q]], fe * exs)
                    plsc.store_scatter(rows_f, [rsp, od_idx[q]], fo * exs)
                return 0

            lax.fori_loop(0, B, _srow, 0)

        def _drain(k):
            pltpu.make_async_copy(rows_f, num_acc.at[didx_v.at[k]],
                                  sem_s0).wait()
            pltpu.make_async_copy(ex_v.at[0], den_acc.at[didx_v.at[k]],
                                  sem_s0).wait()

        def _process(g, k, drain_prev):
            p = k % 2
            cg = g * SCK + k
            pltpu.make_async_copy(h_hbm.at[sidx_v.at[k]], rows_bf[p],
                                  sems_g[p]).wait()
            if drain_prev:
                _drain(k)
            _scale(p, cg)
            pltpu.async_copy(rows_f, num_acc.at[didx_v.at[k]], sem_s0,
                             add=True)
            pltpu.async_copy(ex_v.at[cg], den_acc.at[didx_v.at[k]], sem_s0,
                             add=True)

        def _super_b(g, _):
            @pl.when(g >= 1)
            def _():
                _drain(SCK - 1)

            r0 = wid * CH + g * SCK
            pltpu.sync_copy(src_hbm.at[pl.ds(r0, SCK)], sidx_v)
            pltpu.sync_copy(dst_hbm.at[pl.ds(r0, SCK)], didx_v)
            for k in range(SCK):
                pltpu.async_copy(h_hbm.at[sidx_v.at[k]], rows_bf[k % 2],
                                 sems_g[k % 2])
                if k >= 1:
                    _process(g, k - 1, drain_prev=(k >= 2))
            _process(g, SCK - 1, drain_prev=True)
            return 0

        lax.fori_loop(0, NSUP, _super_b, 0)

        _drain(SCK - 1)

    pl.run_scoped(_phase_b,
                  pltpu.VMEM((B, D), jnp.bfloat16),
                  pltpu.VMEM((B, D), jnp.bfloat16),
                  pltpu.VMEM((B, D), jnp.float32),
                  pltpu.VMEM((SCK, B), jnp.int32),
                  pltpu.VMEM((SCK, B), jnp.int32))

    plsc.subcore_barrier()
    for i in range(4):
        r0 = s * 624 + i * 128
        pltpu.sync_copy(num_acc.at[pl.ds(r0, 128)],
                        num_out.at[c, pl.ds(r0, 128)])
    pltpu.sync_copy(num_acc.at[pl.ds(s * 624 + 512, 112)],
                    num_out.at[c, pl.ds(s * 624 + 512, 112)])

    @pl.when(s == 0)
    def _():
        pltpu.sync_copy(num_acc.at[pl.ds(9984, 16)],
                        num_out.at[c, pl.ds(9984, 16)])

    @pl.when(s < 10)
    def _():
        pltpu.sync_copy(den_acc.at[pl.ds(s * 1024, 1024)],
                        den_out.at[pl.ds(c * NPAD + s * 1024, 1024)])


@functools.cache
def _make_sc_layer():
    return pl.kernel(
        _sc_body,
        out_type=(jax.ShapeDtypeStruct((NC, N, D), jnp.float32),
                  jax.ShapeDtypeStruct((NC * NPAD,), jnp.float32)),
        mesh=plsc.VectorSubcoreMesh(core_axis_name="c", subcore_axis_name="s",
                                    num_cores=NC, num_subcores=NS),
        scratch_types=[
            pltpu.VMEM((CH, B), jnp.float32),
            pltpu.VMEM((1024,), jnp.float32),
            pltpu.VMEM_SHARED((N, D), jnp.float32),
            pltpu.VMEM_SHARED((NPAD,), jnp.float32),
            pltpu.SemaphoreType.DMA,
            pltpu.SemaphoreType.DMA,
            pltpu.SemaphoreType.DMA,
        ],
        compiler_params=pltpu.CompilerParams(use_tc_tiling_on_sc=False,
                                             needs_layout_passes=False),
    )


def _sc_layer(h, es, ed, src_p, dst_p):
    num_p, den_flat = _make_sc_layer()(h, es, ed, src_p, dst_p)
    return num_p, den_flat.reshape(NC, NPAD, 1)


_BLK = 1000
_GRID = N // _BLK


def _tc_first_body(x_ref, w_ref, as_ref, ad_ref, h_ref, es_ref, ed_ref):
    h = jnp.dot(x_ref[...], w_ref[...], preferred_element_type=jnp.float32)
    h_ref[...] = h.astype(jnp.bfloat16)
    es_ref[...] = (h @ as_ref[...])[:, None]
    ed_ref[...] = (h @ ad_ref[...])[:, None]


def _tc_mid_body(np_ref, dp_ref, b_ref, w_ref, as_ref, ad_ref,
                 h_ref, es_ref, ed_ref):
    num = np_ref[0] + np_ref[1]
    den = dp_ref[0, :, 0] + dp_ref[1, :, 0]
    x = jnp.maximum(num / (den + jnp.float32(1e-16))[:, None]
                    + b_ref[...][None, :], 0.0)
    h = jnp.dot(x, w_ref[...], preferred_element_type=jnp.float32)
    h_ref[...] = h.astype(jnp.bfloat16)
    es_ref[...] = (h @ as_ref[...])[:, None]
    ed_ref[...] = (h @ ad_ref[...])[:, None]


def _tc_final_body(np_ref, dp_ref, b_ref, o_ref):
    num = np_ref[0] + np_ref[1]
    den = dp_ref[0, :, 0] + dp_ref[1, :, 0]
    o_ref[...] = (num / (den + jnp.float32(1e-16))[:, None]
                  + b_ref[...][None, :])


_vec_spec = pl.BlockSpec((128,), lambda i: (0,))
_w_spec = pl.BlockSpec((D, D), lambda i: (0, 0))
_den_spec = pl.BlockSpec((NC, _BLK, 1), lambda i: (0, i, 0))
_h_out = [jax.ShapeDtypeStruct((N, D), jnp.bfloat16),
          jax.ShapeDtypeStruct((N, 1), jnp.float32),
          jax.ShapeDtypeStruct((N, 1), jnp.float32)]
_h_specs = [pl.BlockSpec((_BLK, D), lambda i: (i, 0)),
            pl.BlockSpec((_BLK, 1), lambda i: (i, 0)),
            pl.BlockSpec((_BLK, 1), lambda i: (i, 0))]


def _tc_first(x, W, a_s, a_d):
    return pl.pallas_call(
        _tc_first_body,
        grid=(_GRID,),
        in_specs=[pl.BlockSpec((_BLK, D), lambda i: (i, 0)),
                  _w_spec, _vec_spec, _vec_spec],
        out_specs=_h_specs,
        out_shape=_h_out,
    )(x, W, a_s, a_d)


def _tc_mid(num_p, den_p, b, W, a_s, a_d):
    return pl.pallas_call(
        _tc_mid_body,
        grid=(_GRID,),
        in_specs=[pl.BlockSpec((NC, _BLK, D), lambda i: (0, i, 0)),
                  _den_spec,
                  _vec_spec, _w_spec, _vec_spec, _vec_spec],
        out_specs=_h_specs,
        out_shape=_h_out,
    )(num_p, den_p, b, W, a_s, a_d)


def _tc_final(num_p, den_p, b):
    return pl.pallas_call(
        _tc_final_body,
        grid=(_GRID,),
        in_specs=[pl.BlockSpec((NC, _BLK, D), lambda i: (0, i, 0)),
                  _den_spec,
                  _vec_spec],
        out_specs=pl.BlockSpec((_BLK, D), lambda i: (i, 0)),
        out_shape=jax.ShapeDtypeStruct((N, D), jnp.float32),
    )(num_p, den_p, b)


def kernel(x, edge_index, W1, as1, ad1, b1, W2, as2, ad2, b2,
           W3, as3, ad3, b3):
    ei = edge_index.astype(jnp.int32)
    pad = E_PAD - E
    src_p = jnp.concatenate(
        [ei[0], jnp.zeros((pad,), jnp.int32)]).reshape(NW * CH, B)
    dst_p = jnp.concatenate(
        [ei[1], jnp.zeros((pad,), jnp.int32)]).reshape(NW * CH, B)

    h, es, ed = _tc_first(x, W1, as1, ad1)
    num_p, den_p = _sc_layer(h, es.reshape(N), ed.reshape(N), src_p, dst_p)
    h, es, ed = _tc_mid(num_p, den_p, b1, W2, as2, ad2)
    num_p, den_p = _sc_layer(h, es.reshape(N), ed.reshape(N), src_p, dst_p)
    h, es, ed = _tc_mid(num_p, den_p, b2, W3, as3, ad3)
    num_p, den_p = _sc_layer(h, es.reshape(N), ed.reshape(N), src_p, dst_p)
    return _tc_final(num_p, den_p, b3)
